# Initial kernel scaffold; baseline (speedup 1.0000x reference)
import jax
import jax.numpy as jnp
from jax.experimental import pallas as pl


def kernel(x, W):
    raise NotImplementedError("write your pallas kernel here")


# speedup vs baseline: 1.4589x; 1.4589x over previous
import functools

import jax
import jax.numpy as jnp
from jax import lax
from jax.experimental import pallas as pl
from jax.experimental.pallas import tpu as pltpu
from jax.experimental.pallas import tpu_sc as plsc

D = 32
IW = 128
K = 8


@functools.partial(jax.jit, static_argnums=())
def _gather_sc(x2d, W):
    info = plsc.get_sparse_core_info()
    nw = info.num_cores * info.num_subcores
    R = x2d.shape[0]
    rows_per_w = R // nw
    n_chunks = rows_per_w // K

    mesh = plsc.VectorSubcoreMesh(core_axis_name="c", subcore_axis_name="s")

    @functools.partial(
        pl.kernel,
        mesh=mesh,
        out_type=jax.ShapeDtypeStruct((R, IW, D), jnp.float32),
        scratch_types=[
            pltpu.VMEM((K, IW), jnp.int32),
            pltpu.VMEM((K, IW, D), jnp.float32),
            pltpu.SemaphoreType.DMA,
        ],
        compiler_params=pltpu.CompilerParams(use_tc_tiling_on_sc=False),
    )
    def body(x_hbm, w_hbm, out_hbm, idx_v, rows_v, sem):
        wid = lax.axis_index("s") * info.num_cores + lax.axis_index("c")
        w_base = wid * rows_per_w

        def chunk(g, carry):
            row0 = w_base + g * K
            pltpu.sync_copy(x_hbm.at[pl.ds(row0, K)], idx_v)
            copies = [
                pltpu.async_copy(w_hbm.at[idx_v.at[j]], rows_v.at[j], sem)
                for j in range(K)
            ]
            for c in copies:
                c.wait()
            pltpu.sync_copy(rows_v, out_hbm.at[pl.ds(row0, K)])
            return carry

        lax.fori_loop(0, n_chunks, chunk, 0)

    return body(x2d, W)


def kernel(x, W):
    B, L = x.shape
    idx = x.astype(jnp.int32).reshape(-1, IW)
    out = _gather_sc(idx, W)
    return out.reshape(B, L, D)


# speedup vs baseline: 1.4766x; 1.0122x over previous
import functools

import jax
import jax.numpy as jnp
from jax import lax
from jax.experimental import pallas as pl
from jax.experimental.pallas import tpu as pltpu
from jax.experimental.pallas import tpu_sc as plsc

D = 32
NB = 8


def _gather_sc(x, W):
    info = plsc.get_sparse_core_info()
    nw = info.num_cores * info.num_subcores
    B, L = x.shape
    rows_per_w = B // nw
    n_chunks = rows_per_w // NB

    mesh = plsc.VectorSubcoreMesh(core_axis_name="c", subcore_axis_name="s")

    @functools.partial(
        pl.kernel,
        mesh=mesh,
        out_type=jax.ShapeDtypeStruct((B, L, D), jnp.float32),
        scratch_types=[
            pltpu.VMEM((NB, L), jnp.int32),
            pltpu.VMEM((NB, L, D), jnp.float32),
            pltpu.SemaphoreType.DMA,
        ],
        compiler_params=pltpu.CompilerParams(use_tc_tiling_on_sc=False),
    )
    def body(x_hbm, w_hbm, out_hbm, idx_v, rows_v, sem):
        wid = lax.axis_index("s") * info.num_cores + lax.axis_index("c")
        w_base = wid * rows_per_w

        def chunk(g, carry):
            b0 = w_base + g * NB
            pltpu.sync_copy(x_hbm.at[pl.ds(b0, NB)], idx_v)
            copies = []
            for i in range(NB):
                copies.append(pltpu.async_copy(
                    w_hbm.at[idx_v.at[i, pl.ds(0, 128)]],
                    rows_v.at[i, pl.ds(0, 128)], sem))
                copies.append(pltpu.async_copy(
                    w_hbm.at[idx_v.at[i, pl.ds(128, L - 128)]],
                    rows_v.at[i, pl.ds(128, L - 128)], sem))
            for c in copies:
                c.wait()
            pltpu.sync_copy(rows_v, out_hbm.at[pl.ds(b0, NB)])
            return carry

        lax.fori_loop(0, n_chunks, chunk, 0)

    return body(x, W)


def kernel(x, W):
    return _gather_sc(x.astype(jnp.int32), W)
